# Initial kernel scaffold; baseline (speedup 1.0000x reference)
import functools

import jax
import jax.numpy as jnp
from jax.experimental import pallas as pl

_SMOOTH = 0.1
_ROW_BLOCK = 8


def _loss_kernel(pred_ref, tgt_ref, out_ref, *, num_classes, batch):
    x = pred_ref[...]
    t = tgt_ref[...]
    rb = x.shape[0]

    m = jnp.max(x, axis=1, keepdims=True)
    s_sum = jnp.sum(x, axis=1, keepdims=True)
    e_sum = jnp.sum(jnp.exp(x - m), axis=1, keepdims=True)

    cols = jax.lax.broadcasted_iota(jnp.int32, (rb, num_classes), 1)
    pt = jnp.sum(jnp.where(cols == t, x, 0.0), axis=1, keepdims=True)

    lse = jnp.log(e_sum)
    eps = _SMOOTH / (num_classes - 1)
    row_loss = -(
        eps * (s_sum - num_classes * (m + lse))
        + (1.0 - _SMOOTH - eps) * (pt - m - lse)
    )

    @pl.when(pl.program_id(0) == 0)
    def _():
        out_ref[0, 0] = 0.0

    out_ref[0, 0] += jnp.sum(row_loss) / batch


def kernel(pred, target):
    batch, num_classes = pred.shape
    tgt = target.astype(jnp.int32).reshape(batch, 1)
    grid = batch // _ROW_BLOCK

    out = pl.pallas_call(
        functools.partial(_loss_kernel, num_classes=num_classes, batch=batch),
        grid=(grid,),
        in_specs=[
            pl.BlockSpec((_ROW_BLOCK, num_classes), lambda i: (i, 0)),
            pl.BlockSpec((_ROW_BLOCK, 1), lambda i: (i, 0)),
        ],
        out_specs=pl.BlockSpec((1, 1), lambda i: (0, 0)),
        out_shape=jax.ShapeDtypeStruct((1, 1), jnp.float32),
    )(pred, tgt)
    return out[0, 0]


# speedup vs baseline: 1.7047x; 1.7047x over previous
import functools

import jax
import jax.numpy as jnp
from jax.experimental import pallas as pl

_SMOOTH = 0.1
_ROW_BLOCK = 8


def _loss_kernel(pred_ref, tgt_ref, out_ref, *, num_classes, batch):
    x = pred_ref[...]
    t = tgt_ref[...]
    rb = x.shape[0]

    m = jnp.max(x, axis=1, keepdims=True)
    s_sum = jnp.sum(x, axis=1, keepdims=True)
    e_sum = jnp.sum(jnp.exp(x - m), axis=1, keepdims=True)

    cols = jax.lax.broadcasted_iota(jnp.int32, (rb, num_classes), 1)
    pt = jnp.sum(jnp.where(cols == t, x, 0.0), axis=1, keepdims=True)

    lse = jnp.log(e_sum)
    eps = _SMOOTH / (num_classes - 1)
    row_loss = -(
        eps * (s_sum - num_classes * (m + lse))
        + (1.0 - _SMOOTH - eps) * (pt - m - lse)
    )

    @pl.when(pl.program_id(0) == 0)
    def _():
        out_ref[...] = jnp.zeros((1, 1), jnp.float32)

    out_ref[...] += jnp.sum(row_loss).reshape(1, 1) / batch


def kernel(pred, target):
    batch, num_classes = pred.shape
    tgt = target.astype(jnp.int32).reshape(batch, 1)
    grid = batch // _ROW_BLOCK

    out = pl.pallas_call(
        functools.partial(_loss_kernel, num_classes=num_classes, batch=batch),
        grid=(grid,),
        in_specs=[
            pl.BlockSpec((_ROW_BLOCK, num_classes), lambda i: (i, 0)),
            pl.BlockSpec((_ROW_BLOCK, 1), lambda i: (i, 0)),
        ],
        out_specs=pl.BlockSpec((1, 1), lambda i: (0, 0)),
        out_shape=jax.ShapeDtypeStruct((1, 1), jnp.float32),
    )(pred, tgt)
    return out[0, 0]
